# Initial kernel scaffold; baseline (speedup 1.0000x reference)
import jax
import jax.numpy as jnp
from jax.experimental import pallas as pl


def kernel(triples, weights1, weights2, bias1, bias2):
    raise NotImplementedError("write your pallas kernel here")


# speedup vs baseline: 45.4778x; 45.4778x over previous
import functools

import jax
import jax.numpy as jnp
from jax import lax
from jax.experimental import pallas as pl
from jax.experimental.pallas import tpu as pltpu
from jax.experimental.pallas import tpu_sc as plsc

NC, NS, LN = 2, 16, 16


def _cdiv(a, b):
    return (a + b - 1) // b


def _make_edge_pass(cfg, with_hist, interpret=False):
    n = cfg["n"]
    CH = cfg["ch"]
    SUB = cfg["sub"]
    NSUB = CH // SUB
    T_EDGE = cfg["t_edge"]
    T_HIST = T_EDGE * NC
    NCH_L = T_EDGE // CH
    NCH_H = T_HIST // CH
    HBINS = cfg["hbins"]
    HROWS = cfg["hrows"]
    HB_T = HBINS // NS
    HR_T = HROWS // NS
    ZF = cfg["zf"]
    ZR = cfg["zr"]
    emb = cfg["emb"]

    mesh = plsc.VectorSubcoreMesh(
        core_axis_name="c", subcore_axis_name="s", num_cores=NC,
        num_subcores=NS)

    e_pad = NC * NS * T_EDGE

    if with_hist:
        out_type = [
            jax.ShapeDtypeStruct((NC, HROWS, emb), jnp.float32),
            jax.ShapeDtypeStruct((e_pad,), jnp.float32),
        ]
    else:
        out_type = [jax.ShapeDtypeStruct((NC, HROWS, emb), jnp.float32)]

    scratch_types = [
        pltpu.VMEM_SHARED((HBINS,), jnp.float32),
        pltpu.VMEM_SHARED((HROWS, emb), jnp.float32),
        pltpu.VMEM((ZF,), jnp.float32),
        pltpu.VMEM((ZR, emb), jnp.float32),
        pltpu.VMEM((SUB,), jnp.float32),
        pltpu.VMEM((CH,), jnp.int32),
        pltpu.VMEM((CH,), jnp.int32),
        pltpu.VMEM((CH,), jnp.int32),
        pltpu.VMEM((NSUB, SUB), jnp.int32),
        pltpu.VMEM((NSUB, SUB), jnp.int32),
        pltpu.VMEM((NSUB, SUB), jnp.int32),
        pltpu.VMEM((CH,), jnp.float32),
        pltpu.VMEM((CH, emb), jnp.float32),
        pltpu.SemaphoreType.DMA,
    ]

    def body(*refs):
        if with_hist:
            (fr_h, p_h, to_h, tab_h, part_h, vals_h,
             hist, acc, zflat, zrows, ones, frv, pv, tov,
             idxw, idxv, idxf, valsv, rowsv, sem) = refs
        else:
            (fr_h, p_h, to_h, vals_in_h, tab_h, part_h,
             hist, acc, zflat, zrows, ones, frv, pv, tov,
             idxw, idxv, idxf, valsv, rowsv, sem) = refs

        cid = lax.axis_index("c")
        sid = lax.axis_index("s")
        wid = cid * NS + sid

        def zf_body(i, _):
            zflat[pl.ds(i * LN, LN)] = jnp.zeros((LN,), jnp.float32)
            return 0
        lax.fori_loop(0, ZF // LN, zf_body, 0)

        def zr_body(i, _):
            zrows[i] = jnp.zeros((LN,), jnp.float32)
            return 0
        lax.fori_loop(0, ZR, zr_body, 0)

        for i in range(SUB // LN):
            ones[pl.ds(i * LN, LN)] = jnp.ones((LN,), jnp.float32)

        if with_hist:
            for k in range(HB_T // ZF):
                pltpu.sync_copy(zflat, hist.at[pl.ds(sid * HB_T + k * ZF, ZF)])
        for k in range(HR_T // ZR):
            pltpu.sync_copy(zrows, acc.at[pl.ds(sid * HR_T + k * ZR, ZR)])

        plsc.subcore_barrier()

        if with_hist:
            hbase = sid * T_HIST

            def hist_body(k, _):
                off = hbase + k * CH
                pltpu.sync_copy(fr_h.at[pl.ds(off, CH)], frv)
                pltpu.sync_copy(p_h.at[pl.ds(off, CH)], pv)
                for j in range(CH // LN):
                    r, col = j * LN // SUB, (j * LN) % SUB
                    pj = pv[pl.ds(j * LN, LN)]
                    fj = frv[pl.ds(j * LN, LN)]
                    idxv[r, pl.ds(col, LN)] = pj * n + fj
                for r in range(NSUB):
                    pltpu.sync_copy(ones, hist.at[idxv.at[r]], add=True)
                return 0
            lax.fori_loop(0, NCH_H, hist_body, 0)
            plsc.subcore_barrier()

        ebase = wid * T_EDGE

        def edge_body(k, _):
            off = ebase + k * CH
            pltpu.sync_copy(fr_h.at[pl.ds(off, CH)], frv)
            pltpu.sync_copy(p_h.at[pl.ds(off, CH)], pv)
            pltpu.sync_copy(to_h.at[pl.ds(off, CH)], tov)
            if not with_hist:
                pltpu.sync_copy(vals_in_h.at[pl.ds(off, CH)], valsv)
            for j in range(CH // LN):
                r, col = j * LN // SUB, (j * LN) % SUB
                pj = pv[pl.ds(j * LN, LN)]
                fj = frv[pl.ds(j * LN, LN)]
                tj = tov[pl.ds(j * LN, LN)]
                idxw[r, pl.ds(col, LN)] = pj * n + tj
                idxf[r, pl.ds(col, LN)] = fj
                if with_hist:
                    idxv[r, pl.ds(col, LN)] = pj * n + fj

            cps = []
            for r in range(NSUB):
                cps.append(pltpu.async_copy(
                    tab_h.at[idxw.at[r]], rowsv.at[pl.ds(r * SUB, SUB)], sem))
            for cp in cps:
                cp.wait()

            if with_hist:
                cps = []
                for r in range(NSUB):
                    cps.append(pltpu.async_copy(
                        hist.at[idxv.at[r]],
                        valsv.at[pl.ds(r * SUB, SUB)], sem))
                for cp in cps:
                    cp.wait()
                for j in range(CH // LN):
                    valsv[pl.ds(j * LN, LN)] = (
                        1.0 / valsv[pl.ds(j * LN, LN)])
                pltpu.sync_copy(valsv, vals_h.at[pl.ds(off, CH)])

            def scale_body(i, _):
                vv = valsv[pl.ds(i * LN, LN)]
                for l in range(LN):
                    q = i * LN + l
                    rowsv[q] = rowsv[q] * jnp.full((LN,), vv[l], jnp.float32)
                return 0
            lax.fori_loop(0, CH // LN, scale_body, 0)

            for r in range(NSUB):
                pltpu.sync_copy(rowsv.at[pl.ds(r * SUB, SUB)],
                                acc.at[idxf.at[r]], add=True)
            return 0
        lax.fori_loop(0, NCH_L, edge_body, 0)

        plsc.subcore_barrier()

        pltpu.sync_copy(acc.at[pl.ds(sid * HR_T, HR_T)],
                        part_h.at[cid].at[pl.ds(sid * HR_T, HR_T)])

    kern = pl.kernel(body, out_type=out_type, mesh=mesh,
                     scratch_types=scratch_types, interpret=interpret,
                     compiler_params=pltpu.CompilerParams(
                         use_tc_tiling_on_sc=False))
    return kern


def _tc_hw2(n_real, r, emb, ncls, blk, interpret=False):
    grid = n_real // blk

    def body(hp_ref, b1_ref, w2_ref, out_ref):
        h = jax.nn.relu(hp_ref[0] + hp_ref[1] + b1_ref[0][None, :])
        for p in range(r):
            out_ref[p] = jnp.dot(h, w2_ref[p],
                                 preferred_element_type=jnp.float32)

    return pl.pallas_call(
        body,
        grid=(grid,),
        in_specs=[
            pl.BlockSpec((NC, blk, emb), lambda i: (0, i, 0)),
            pl.BlockSpec((1, emb), lambda i: (0, 0)),
            pl.BlockSpec((r, emb, ncls), lambda i: (0, 0, 0)),
        ],
        out_specs=pl.BlockSpec((r, blk, ncls), lambda i: (0, i, 0)),
        out_shape=jax.ShapeDtypeStruct((r, n_real, ncls), jnp.float32),
        interpret=interpret,
    )


def _tc_combine(n_real, ncls, blk, interpret=False):
    grid = n_real // blk

    def body(op_ref, b2_ref, out_ref):
        out_ref[...] = op_ref[0] + op_ref[1] + b2_ref[0][None, :]

    return pl.pallas_call(
        body,
        grid=(grid,),
        in_specs=[
            pl.BlockSpec((NC, blk, ncls), lambda i: (0, i, 0)),
            pl.BlockSpec((1, ncls), lambda i: (0, 0)),
        ],
        out_specs=pl.BlockSpec((blk, ncls), lambda i: (i, 0)),
        out_shape=jax.ShapeDtypeStruct((n_real, ncls), jnp.float32),
        interpret=interpret,
    )


def _rgcn(triples, weights1, weights2, bias1, bias2, cfg, interpret=False):
    n, r = cfg["n"], cfg["r"]
    emb, ncls = cfg["emb"], cfg["ncls"]
    e_pad = NC * NS * cfg["t_edge"]
    e_real = triples.shape[0]
    npad = e_pad - e_real

    fr = jnp.concatenate(
        [triples[:, 0], jnp.full((npad,), n, jnp.int32)])
    p = jnp.concatenate(
        [triples[:, 1], jnp.full((npad,), r - 1, jnp.int32)])
    to = jnp.concatenate(
        [triples[:, 2], jnp.zeros((npad,), jnp.int32)])

    w1_flat = weights1.reshape(r * n, emb)

    hpart, vals = _make_edge_pass(cfg, with_hist=True, interpret=interpret)(
        fr, p, to, w1_flat)

    hw2 = _tc_hw2(n, r, emb, ncls, cfg["tc_blk"], interpret=interpret)(
        hpart[:, :n, :], bias1.reshape(1, emb), weights2)
    hw2_flat = hw2.reshape(r * n, ncls)

    (opart,) = _make_edge_pass(cfg, with_hist=False, interpret=interpret)(
        fr, p, to, vals, hw2_flat)

    out = _tc_combine(n, ncls, cfg["tc_blk"], interpret=interpret)(
        opart[:, :n, :], bias2.reshape(1, ncls))
    return out


_CFG_FULL = dict(
    n=50000, r=8, emb=16, ncls=16,
    ch=1024, sub=128, t_edge=50176,
    hbins=409600, hrows=50048,
    zf=3200, zr=782,
    tc_blk=2000,
)


def kernel(triples, weights1, weights2, bias1, bias2):
    return _rgcn(triples, weights1, weights2, bias1, bias2, _CFG_FULL)
